# baseline (speedup 1.0000x reference)
import functools

import jax
import jax.numpy as jnp
from jax import lax
from jax.experimental import pallas as pl
from jax.experimental.pallas import tpu as pltpu
from jax.experimental.pallas import tpu_sc as plsc

_SEP = 102
_BS = 1024


def _bounds_sparsecore(input_ids):
    B, S = input_ids.shape
    info = plsc.get_sparse_core_info()
    nc = info.num_cores
    mesh = plsc.VectorSubcoreMesh(core_axis_name="c", subcore_axis_name="s")

    @functools.partial(
        pl.kernel,
        mesh=mesh,
        out_type=jax.ShapeDtypeStruct((B, 16), jnp.int32),
        scratch_types=[
            pltpu.VMEM((S,), jnp.int32),
            pltpu.VMEM((16,), jnp.int32),
        ],
    )
    def sc_kernel(ids_hbm, out_hbm, ids_v, res_v):
        wid = lax.axis_index("s") * nc + lax.axis_index("c")

        @pl.when(wid < B)
        def _():
            pltpu.sync_copy(ids_hbm.at[wid], ids_v)

            def step(i, carry):
                mn, mx, ct = carry
                v = ids_v[pl.ds(i * 16, 16)]
                pos = i * 16 + lax.iota(jnp.int32, 16)
                is_sep = v == _SEP
                mn = jnp.minimum(mn, jnp.where(is_sep, pos, S))
                mx = jnp.maximum(mx, jnp.where(is_sep, pos, -1))
                ct = ct + jnp.where(is_sep, 1, 0)
                return mn, mx, ct

            init = (
                jnp.full((16,), S, jnp.int32),
                jnp.full((16,), -1, jnp.int32),
                jnp.zeros((16,), jnp.int32),
            )
            mn, mx, ct = lax.fori_loop(0, S // 16, step, init)
            fs = mn[0]
            ls = mx[0]
            c = ct[0]
            for j in range(1, 16):
                fs = jnp.minimum(fs, mn[j])
                ls = jnp.maximum(ls, mx[j])
                c = c + ct[j]
            lane = lax.iota(jnp.int32, 16)
            res = jnp.where(
                lane == 0, fs, jnp.where(lane == 1, ls, jnp.where(lane == 2, c, 0))
            )
            res_v[...] = res
            pltpu.sync_copy(res_v, out_hbm.at[wid])

    return sc_kernel(input_ids)


_CH = 512
_NBUF = 5


def _pool_body(bounds_ref, hs_hbm, out_ref, buf, sems, *, S, CH, NBUF):
    b = pl.program_id(0)
    fs = bounds_ref[b, 0]
    ls = bounds_ref[b, 1]
    cnt = bounds_ref[b, 2]

    t_end = jnp.where(cnt >= 1, fs, 0)
    l_end = jnp.where(cnt >= 2, ls, S)
    t_d = jnp.maximum(t_end - 1, 1)
    l_d = jnp.maximum(l_end - fs - 1, 1)
    inv_t = 1.0 / t_d.astype(jnp.float32)
    inv_l = 1.0 / l_d.astype(jnp.float32)

    needed_end = jnp.where(cnt >= 1, jnp.where(cnt >= 2, ls, S), 0)
    nb = (needed_end + CH - 1) // CH

    out_ref[...] = jnp.zeros_like(out_ref)

    def dma(j, slot):
        return pltpu.make_async_copy(
            hs_hbm.at[b, pl.ds(j * CH, CH), :], buf.at[slot], sems.at[slot]
        )

    def issue(j, _):
        dma(j, j % NBUF).start()
        return 0

    lax.fori_loop(0, jnp.minimum(nb, NBUF), issue, 0)

    zero = jnp.zeros((), jnp.float32)
    sub = lax.broadcasted_iota(jnp.int32, (CH, 1), 0)

    def step(j, _):
        slot = j % NBUF
        dma(j, slot).wait()
        pos = j * CH + sub
        mt = (pos >= 1) & (pos < t_end)
        ml = (pos > fs) & (pos < l_end)
        x = buf[slot]
        t_part = jnp.sum(jnp.where(mt, x, zero), axis=0) * inv_t
        l_part = jnp.sum(jnp.where(ml, x, zero), axis=0) * inv_l
        out_ref[0, 0, :] += t_part
        out_ref[0, 1, :] += l_part

        @pl.when(j + NBUF < nb)
        def _():
            dma(j + NBUF, slot).start()

        return 0

    lax.fori_loop(0, nb, step, 0)


def _pool_tc(hidden_states, bounds, *, interpret=False):
    B, S, H = hidden_states.shape
    CH, NBUF = _CH, _NBUF

    grid_spec = pltpu.PrefetchScalarGridSpec(
        num_scalar_prefetch=1,
        grid=(B,),
        in_specs=[pl.BlockSpec(memory_space=pl.ANY)],
        out_specs=pl.BlockSpec((1, 2, H), lambda b, bounds_ref: (b, 0, 0)),
        scratch_shapes=[
            pltpu.VMEM((NBUF, CH, H), jnp.float32),
            pltpu.SemaphoreType.DMA((NBUF,)),
        ],
    )
    out = pl.pallas_call(
        functools.partial(_pool_body, S=S, CH=CH, NBUF=NBUF),
        grid_spec=grid_spec,
        out_shape=jax.ShapeDtypeStruct((B, 2, H), jnp.float32),
        interpret=interpret,
    )(bounds, hidden_states)
    return out[:, 0, :], out[:, 1, :]


@jax.jit
def kernel(hidden_states, input_ids):
    bounds = _bounds_sparsecore(input_ids)
    return _pool_tc(hidden_states, bounds)


# speedup vs baseline: 1.1023x; 1.1023x over previous
import functools

import jax
import jax.numpy as jnp
from jax import lax
from jax.experimental import pallas as pl
from jax.experimental.pallas import tpu as pltpu
from jax.experimental.pallas import tpu_sc as plsc

_SEP = 102
_BS = 1024


def _bounds_sparsecore(input_ids):
    B, S = input_ids.shape
    info = plsc.get_sparse_core_info()
    nc = info.num_cores
    mesh = plsc.VectorSubcoreMesh(core_axis_name="c", subcore_axis_name="s")

    @functools.partial(
        pl.kernel,
        mesh=mesh,
        out_type=jax.ShapeDtypeStruct((B, 16), jnp.int32),
        scratch_types=[
            pltpu.VMEM((S,), jnp.int32),
            pltpu.VMEM((16,), jnp.int32),
        ],
    )
    def sc_kernel(ids_hbm, out_hbm, ids_v, res_v):
        wid = lax.axis_index("s") * nc + lax.axis_index("c")

        @pl.when(wid < B)
        def _():
            pltpu.sync_copy(ids_hbm.at[wid], ids_v)

            def step(i, carry):
                mn, mx, ct = carry
                v = ids_v[pl.ds(i * 16, 16)]
                pos = i * 16 + lax.iota(jnp.int32, 16)
                is_sep = v == _SEP
                mn = jnp.minimum(mn, jnp.where(is_sep, pos, S))
                mx = jnp.maximum(mx, jnp.where(is_sep, pos, -1))
                ct = ct + jnp.where(is_sep, 1, 0)
                return mn, mx, ct

            init = (
                jnp.full((16,), S, jnp.int32),
                jnp.full((16,), -1, jnp.int32),
                jnp.zeros((16,), jnp.int32),
            )
            mn, mx, ct = lax.fori_loop(0, S // 16, step, init)
            fs = mn[0]
            ls = mx[0]
            c = ct[0]
            for j in range(1, 16):
                fs = jnp.minimum(fs, mn[j])
                ls = jnp.maximum(ls, mx[j])
                c = c + ct[j]
            lane = lax.iota(jnp.int32, 16)
            res = jnp.where(
                lane == 0, fs, jnp.where(lane == 1, ls, jnp.where(lane == 2, c, 0))
            )
            res_v[...] = res
            pltpu.sync_copy(res_v, out_hbm.at[wid])

    return sc_kernel(input_ids)


def _pool_body(bounds_ref, hs_ref, out_ref, *, S, BS):
    b = pl.program_id(0)
    s = pl.program_id(1)
    fs = bounds_ref[b, 0]
    ls = bounds_ref[b, 1]
    cnt = bounds_ref[b, 2]

    t_end = jnp.where(cnt >= 1, fs, 0)
    l_end = jnp.where(cnt >= 2, ls, S)
    t_d = jnp.maximum(t_end - 1, 1)
    l_d = jnp.maximum(l_end - fs - 1, 1)
    inv_t = 1.0 / t_d.astype(jnp.float32)
    inv_l = 1.0 / l_d.astype(jnp.float32)

    needed_end = jnp.where(cnt >= 1, jnp.where(cnt >= 2, ls, S), 0)

    @pl.when(s == 0)
    def _():
        out_ref[...] = jnp.zeros_like(out_ref)

    @pl.when(s * BS < needed_end)
    def _():
        pos = s * BS + lax.broadcasted_iota(jnp.int32, (BS, 1), 0)
        mt = (pos >= 1) & (pos < t_end)
        ml = (pos > fs) & (pos < l_end)

        wt = jnp.where(mt, inv_t, 0.0).reshape(1, BS)
        wl = jnp.where(ml, inv_l, 0.0).reshape(1, BS)
        w = jnp.concatenate([wt, wl], axis=0)
        contrib = lax.dot_general(
            w,
            hs_ref[0],
            (((1,), (0,)), ((), ())),
            precision=lax.Precision.DEFAULT,
            preferred_element_type=jnp.float32,
        )
        out_ref[0] += contrib


def _pool_tc(hidden_states, bounds, *, interpret=False):
    B, S, H = hidden_states.shape
    BS = _BS

    def hs_index(b, s, bounds_ref):
        fs = bounds_ref[b, 0]
        ls = bounds_ref[b, 1]
        cnt = bounds_ref[b, 2]
        needed_end = jnp.where(cnt >= 1, jnp.where(cnt >= 2, ls, S), 0)
        nb = jnp.maximum((needed_end + BS - 1) // BS, 1)
        return (b, jnp.minimum(s, nb - 1), 0)

    grid_spec = pltpu.PrefetchScalarGridSpec(
        num_scalar_prefetch=1,
        grid=(B, S // BS),
        in_specs=[pl.BlockSpec((1, BS, H), hs_index)],
        out_specs=pl.BlockSpec((1, 2, H), lambda b, s, bounds_ref: (b, 0, 0)),
    )
    out = pl.pallas_call(
        functools.partial(_pool_body, S=S, BS=BS),
        grid_spec=grid_spec,
        out_shape=jax.ShapeDtypeStruct((B, 2, H), jnp.float32),
        interpret=interpret,
    )(bounds, hidden_states)
    return out[:, 0, :], out[:, 1, :]


@jax.jit
def kernel(hidden_states, input_ids):
    bounds = _bounds_sparsecore(input_ids)
    return _pool_tc(hidden_states, bounds)
